# baseline (speedup 1.0000x reference)
import jax
import jax.numpy as jnp
from jax import lax
from jax.experimental import pallas as pl
from jax.experimental.pallas import tpu as pltpu
from jax.experimental.pallas import tpu_sc as plsc

_INFO = plsc.get_sparse_core_info()
_NC, _NS = _INFO.num_cores, _INFO.num_subcores
_NW = _NC * _NS

_B = 4
_CW = 2048
_E = 1024
_ROWS = _B * _CW
_TPW = _CW // _NW
_C = 16
_NB = 4
_NCHUNK = (_TPW // _C) * _B
_PC = 32
_LANE = 16
_COLS = _E // _LANE


def _emb_body(x_hbm, wte_hbm, wpe_hbm, out_hbm,
              pos_v, toks, idx_v, gsems, ssems, sidx):
    wid = lax.axis_index("s") * _NC + lax.axis_index("c")
    t0 = wid * _TPW

    def out_off(i):
        return (i & 3) * _CW + t0 + (i >> 2) * _C

    def idx_off(i):
        return (i & 3) * _TPW + (i >> 2) * _C

    descs = [
        pltpu.async_copy(
            x_hbm.at[pl.ds(b * _CW + t0, _TPW)],
            idx_v.at[pl.ds(b * _TPW, _TPW)],
            sidx,
        )
        for b in range(_B)
    ]
    for d in descs:
        d.wait()

    def start_gather(i, s):
        pltpu.async_copy(
            wte_hbm.at[idx_v.at[pl.ds(idx_off(i), _C)]], toks[s], gsems[s])

    start_gather(0, 0)
    pltpu.sync_copy(wpe_hbm.at[pl.ds(t0, _PC)], pos_v)

    def group(g, carry):
        for s in range(_NB):
            i = _NB * g + s
            nxt = (s + 1) % _NB
            @pl.when(i >= _NB - 1)
            def _():
                pltpu.make_async_copy(
                    toks[nxt],
                    out_hbm.at[pl.ds(out_off(i - (_NB - 1)), _C)],
                    ssems[nxt],
                ).wait()

            @pl.when(i <= _NCHUNK - 2)
            def _():
                start_gather(i + 1, nxt)

            pltpu.make_async_copy(
                wte_hbm.at[idx_v.at[pl.ds(idx_off(i), _C)]], toks[s],
                gsems[s]).wait()

            @pl.when(i == _NCHUNK // 2)
            def _():
                pltpu.sync_copy(wpe_hbm.at[pl.ds(t0 + _PC, _PC)], pos_v)

            tok = toks[s]
            pbase = (lax.shift_right_logical(i, 2) & (_PC // _C - 1)) * _C

            @plsc.parallel_loop(0, _C, unroll=8)
            def add_row(r, tok=tok, pbase=pbase):
                for j in range(_COLS):
                    plsc.addupdate(tok.at[r, pl.ds(j * _LANE, _LANE)],
                                   pos_v[pbase + r, pl.ds(j * _LANE, _LANE)])

            pltpu.async_copy(tok, out_hbm.at[pl.ds(out_off(i), _C)], ssems[s])
        return carry

    lax.fori_loop(0, _NCHUNK // _NB, group, 0)
    for i in range(_NCHUNK - (_NB - 1), _NCHUNK):
        pltpu.make_async_copy(
            toks[i % _NB], out_hbm.at[pl.ds(out_off(i), _C)], ssems[i % _NB]
        ).wait()


def _emb_wrapped(x_hbm, wte_hbm, wpe_hbm, out_hbm, pos_v,
                 t0, t1, t2, t3, idx_v,
                 g0, g1, g2, g3,
                 s0, s1, s2, s3, sidx):
    _emb_body(x_hbm, wte_hbm, wpe_hbm, out_hbm, pos_v,
              (t0, t1, t2, t3), idx_v,
              (g0, g1, g2, g3),
              (s0, s1, s2, s3), sidx)


@jax.jit
def _emb(x_flat, wte, wpe):
    mesh = plsc.VectorSubcoreMesh(core_axis_name="c", subcore_axis_name="s")
    return pl.kernel(
        _emb_wrapped,
        out_type=jax.ShapeDtypeStruct((_ROWS, _E), jnp.float32),
        mesh=mesh,
        scratch_types=(
            [pltpu.VMEM((_PC, _E), jnp.float32)]
            + [pltpu.VMEM((_C, _E), jnp.float32) for _ in range(_NB)]
            + [pltpu.VMEM((_B * _TPW,), jnp.int32)]
            + [pltpu.SemaphoreType.DMA for _ in range(2 * _NB + 1)]
        ),
    )(x_flat, wte, wpe)


def kernel(x, wte, wpe, pos):
    del pos
    out = _emb(x.reshape(_ROWS).astype(jnp.int32), wte, wpe)
    return out.reshape(_B, _CW, _E)


# speedup vs baseline: 1.0198x; 1.0198x over previous
import jax
import jax.numpy as jnp
from jax import lax
from jax.experimental import pallas as pl
from jax.experimental.pallas import tpu as pltpu
from jax.experimental.pallas import tpu_sc as plsc

_INFO = plsc.get_sparse_core_info()
_NC, _NS = _INFO.num_cores, _INFO.num_subcores
_NW = _NC * _NS

_B = 4
_CW = 2048
_E = 1024
_ROWS = _B * _CW
_TPW = _CW // _NW
_C = 16
_NB = 4
_NCHUNK = (_TPW // _C) * _B
_PC = 32
_LANE = 16
_COLS = _E // _LANE


def _emb_body(x_hbm, wte_hbm, wpe_hbm, out_hbm,
              pos_v, toks, idx_v, gsems, ssems, sidx):
    wid = lax.axis_index("s") * _NC + lax.axis_index("c")
    t0 = wid * _TPW

    def out_off(i):
        return (i & 3) * _CW + t0 + (i >> 2) * _C

    def idx_off(i):
        return (i & 3) * _TPW + (i >> 2) * _C

    descs = [
        pltpu.async_copy(
            x_hbm.at[pl.ds(b * _CW + t0, _TPW)],
            idx_v.at[pl.ds(b * _TPW, _TPW)],
            sidx,
        )
        for b in range(_B)
    ]
    for d in descs:
        d.wait()

    def start_gather(i, s):
        pltpu.async_copy(
            wte_hbm.at[idx_v.at[pl.ds(idx_off(i), _C)]], toks[s], gsems[s])

    start_gather(0, 0)
    pltpu.sync_copy(wpe_hbm.at[pl.ds(t0, _PC)], pos_v)

    def group(g, carry):
        for s in range(_NB):
            i = _NB * g + s
            nxt = (s + 1) % _NB
            @pl.when(i >= _NB - 1)
            def _():
                pltpu.make_async_copy(
                    toks[nxt],
                    out_hbm.at[pl.ds(out_off(i - (_NB - 1)), _C)],
                    ssems[nxt],
                ).wait()

            @pl.when(i <= _NCHUNK - 2)
            def _():
                start_gather(i + 1, nxt)

            pltpu.make_async_copy(
                wte_hbm.at[idx_v.at[pl.ds(idx_off(i), _C)]], toks[s],
                gsems[s]).wait()

            @pl.when(i == _NCHUNK // 2)
            def _():
                pltpu.sync_copy(wpe_hbm.at[pl.ds(t0 + _PC, _PC)], pos_v)

            tok = toks[s]
            pbase = (lax.shift_right_logical(i, 2) & (_PC // _C - 1)) * _C

            @plsc.parallel_loop(0, _C, unroll=2)
            def add_row(r, tok=tok, pbase=pbase):
                for j in range(_COLS):
                    plsc.addupdate(tok.at[r, pl.ds(j * _LANE, _LANE)],
                                   pos_v[pbase + r, pl.ds(j * _LANE, _LANE)])

            pltpu.async_copy(tok, out_hbm.at[pl.ds(out_off(i), _C)], ssems[s])
        return carry

    lax.fori_loop(0, _NCHUNK // _NB, group, 0)
    for i in range(_NCHUNK - (_NB - 1), _NCHUNK):
        pltpu.make_async_copy(
            toks[i % _NB], out_hbm.at[pl.ds(out_off(i), _C)], ssems[i % _NB]
        ).wait()


def _emb_wrapped(x_hbm, wte_hbm, wpe_hbm, out_hbm, pos_v,
                 t0, t1, t2, t3, idx_v,
                 g0, g1, g2, g3,
                 s0, s1, s2, s3, sidx):
    _emb_body(x_hbm, wte_hbm, wpe_hbm, out_hbm, pos_v,
              (t0, t1, t2, t3), idx_v,
              (g0, g1, g2, g3),
              (s0, s1, s2, s3), sidx)


@jax.jit
def _emb(x_flat, wte, wpe):
    mesh = plsc.VectorSubcoreMesh(core_axis_name="c", subcore_axis_name="s")
    return pl.kernel(
        _emb_wrapped,
        out_type=jax.ShapeDtypeStruct((_ROWS, _E), jnp.float32),
        mesh=mesh,
        scratch_types=(
            [pltpu.VMEM((_PC, _E), jnp.float32)]
            + [pltpu.VMEM((_C, _E), jnp.float32) for _ in range(_NB)]
            + [pltpu.VMEM((_B * _TPW,), jnp.int32)]
            + [pltpu.SemaphoreType.DMA for _ in range(2 * _NB + 1)]
        ),
    )(x_flat, wte, wpe)


def kernel(x, wte, wpe, pos):
    del pos
    out = _emb(x.reshape(_ROWS).astype(jnp.int32), wte, wpe)
    return out.reshape(_B, _CW, _E)


# speedup vs baseline: 1.4093x; 1.3819x over previous
import jax
import jax.numpy as jnp
from jax import lax
from jax.experimental import pallas as pl
from jax.experimental.pallas import tpu as pltpu
from jax.experimental.pallas import tpu_sc as plsc

_INFO = plsc.get_sparse_core_info()
_NC, _NS = _INFO.num_cores, _INFO.num_subcores
_NW = _NC * _NS

_B = 4
_CW = 2048
_E = 1024
_ROWS = _B * _CW
_TPW = _CW // _NW
_C = 16
_NB = 4
_NCHUNK = (_TPW // _C) * _B
_PC = 32
_LANE = 16
_COLS = _E // _LANE


def _emb_body(x_hbm, wte_hbm, wpe_hbm, out_hbm,
              pos_v, toks, idx_v, gsems, ssems, sidx):
    wid = lax.axis_index("s") * _NC + lax.axis_index("c")
    t0 = wid * _TPW

    def out_off(i):
        return (i & 3) * _CW + t0 + (i >> 2) * _C

    def idx_off(i):
        return (i & 3) * _TPW + (i >> 2) * _C

    descs = [
        pltpu.async_copy(
            x_hbm.at[pl.ds(b * _CW + t0, _TPW)],
            idx_v.at[pl.ds(b * _TPW, _TPW)],
            sidx,
        )
        for b in range(_B)
    ]
    for d in descs:
        d.wait()

    def start_gather(i, s):
        pltpu.async_copy(
            wte_hbm.at[idx_v.at[pl.ds(idx_off(i), _C)]], toks[s], gsems[s])

    start_gather(0, 0)
    pltpu.sync_copy(wpe_hbm.at[pl.ds(t0, _PC)], pos_v)

    def group(g, carry):
        for s in range(_NB):
            i = _NB * g + s
            nxt = (s + 1) % _NB
            @pl.when(i >= _NB - 1)
            def _():
                pltpu.make_async_copy(
                    toks[nxt],
                    out_hbm.at[pl.ds(out_off(i - (_NB - 1)), _C)],
                    ssems[nxt],
                ).wait()

            @pl.when(i <= _NCHUNK - 2)
            def _():
                start_gather(i + 1, nxt)

            pltpu.make_async_copy(
                wte_hbm.at[idx_v.at[pl.ds(idx_off(i), _C)]], toks[s],
                gsems[s]).wait()

            @pl.when(i == _NCHUNK // 2)
            def _():
                pltpu.sync_copy(wpe_hbm.at[pl.ds(t0 + _PC, _PC)], pos_v)

            tok = toks[s]
            pbase = (lax.shift_right_logical(i, 2) & (_PC // _C - 1)) * _C

            @plsc.parallel_loop(0, _C * _COLS, unroll=16)
            def add_k(k, tok=tok, pbase=pbase):
                r = lax.shift_right_logical(k, 6)
                j = (k & (_COLS - 1)) * _LANE
                plsc.addupdate(tok.at[r, pl.ds(j, _LANE)],
                               pos_v[pbase + r, pl.ds(j, _LANE)])

            pltpu.async_copy(tok, out_hbm.at[pl.ds(out_off(i), _C)], ssems[s])
        return carry

    lax.fori_loop(0, _NCHUNK // _NB, group, 0)
    for i in range(_NCHUNK - (_NB - 1), _NCHUNK):
        pltpu.make_async_copy(
            toks[i % _NB], out_hbm.at[pl.ds(out_off(i), _C)], ssems[i % _NB]
        ).wait()


def _emb_wrapped(x_hbm, wte_hbm, wpe_hbm, out_hbm, pos_v,
                 t0, t1, t2, t3, idx_v,
                 g0, g1, g2, g3,
                 s0, s1, s2, s3, sidx):
    _emb_body(x_hbm, wte_hbm, wpe_hbm, out_hbm, pos_v,
              (t0, t1, t2, t3), idx_v,
              (g0, g1, g2, g3),
              (s0, s1, s2, s3), sidx)


@jax.jit
def _emb(x_flat, wte, wpe):
    mesh = plsc.VectorSubcoreMesh(core_axis_name="c", subcore_axis_name="s")
    return pl.kernel(
        _emb_wrapped,
        out_type=jax.ShapeDtypeStruct((_ROWS, _E), jnp.float32),
        mesh=mesh,
        scratch_types=(
            [pltpu.VMEM((_PC, _E), jnp.float32)]
            + [pltpu.VMEM((_C, _E), jnp.float32) for _ in range(_NB)]
            + [pltpu.VMEM((_B * _TPW,), jnp.int32)]
            + [pltpu.SemaphoreType.DMA for _ in range(2 * _NB + 1)]
        ),
    )(x_flat, wte, wpe)


def kernel(x, wte, wpe, pos):
    del pos
    out = _emb(x.reshape(_ROWS).astype(jnp.int32), wte, wpe)
    return out.reshape(_B, _CW, _E)


# speedup vs baseline: 1.4688x; 1.0422x over previous
import jax
import jax.numpy as jnp
from jax import lax
from jax.experimental import pallas as pl
from jax.experimental.pallas import tpu as pltpu
from jax.experimental.pallas import tpu_sc as plsc

_INFO = plsc.get_sparse_core_info()
_NC, _NS = _INFO.num_cores, _INFO.num_subcores
_NW = _NC * _NS

_B = 4
_CW = 2048
_E = 1024
_ROWS = _B * _CW
_TPW = _CW // _NW
_C = 16
_NB = 4
_NCHUNK = (_TPW // _C) * _B
_PC = 32
_LANE = 16
_COLS = _E // _LANE


def _emb_body(x_hbm, wte_hbm, wpe_hbm, out_hbm,
              pos_v, toks, idx_v, gsems, ssems, sidx):
    wid = lax.axis_index("s") * _NC + lax.axis_index("c")
    t0 = wid * _TPW

    def out_off(i):
        return (i & 3) * _CW + t0 + (i >> 2) * _C

    def idx_off(i):
        return (i & 3) * _TPW + (i >> 2) * _C

    descs = [
        pltpu.async_copy(
            x_hbm.at[pl.ds(b * _CW + t0, _TPW)],
            idx_v.at[pl.ds(b * _TPW, _TPW)],
            sidx,
        )
        for b in range(_B)
    ]
    for d in descs:
        d.wait()

    def start_gather(i, s):
        pltpu.async_copy(
            wte_hbm.at[idx_v.at[pl.ds(idx_off(i), _C)]], toks[s], gsems[s])

    _AH = 2

    start_gather(0, 0)
    start_gather(1, 1)
    pltpu.sync_copy(wpe_hbm.at[pl.ds(t0, _PC)], pos_v)

    def group(g, carry):
        for s in range(_NB):
            i = _NB * g + s
            nxt = (s + _AH) % _NB
            @pl.when(i >= _AH)
            def _():
                pltpu.make_async_copy(
                    toks[nxt],
                    out_hbm.at[pl.ds(out_off(i - _AH), _C)],
                    ssems[nxt],
                ).wait()

            @pl.when(i <= _NCHUNK - 1 - _AH)
            def _():
                start_gather(i + _AH, nxt)

            pltpu.make_async_copy(
                wte_hbm.at[idx_v.at[pl.ds(idx_off(i), _C)]], toks[s],
                gsems[s]).wait()

            @pl.when(i == _NCHUNK // 2)
            def _():
                pltpu.sync_copy(wpe_hbm.at[pl.ds(t0 + _PC, _PC)], pos_v)

            tok = toks[s]
            pbase = (lax.shift_right_logical(i, 2) & (_PC // _C - 1)) * _C

            @plsc.parallel_loop(0, _C * _COLS, unroll=16)
            def add_k(k, tok=tok, pbase=pbase):
                r = lax.shift_right_logical(k, 6)
                j = (k & (_COLS - 1)) * _LANE
                plsc.addupdate(tok.at[r, pl.ds(j, _LANE)],
                               pos_v[pbase + r, pl.ds(j, _LANE)])

            pltpu.async_copy(tok, out_hbm.at[pl.ds(out_off(i), _C)], ssems[s])
        return carry

    lax.fori_loop(0, _NCHUNK // _NB, group, 0)
    for i in range(_NCHUNK - _AH, _NCHUNK):
        pltpu.make_async_copy(
            toks[i % _NB], out_hbm.at[pl.ds(out_off(i), _C)], ssems[i % _NB]
        ).wait()


def _emb_wrapped(x_hbm, wte_hbm, wpe_hbm, out_hbm, pos_v,
                 t0, t1, t2, t3, idx_v,
                 g0, g1, g2, g3,
                 s0, s1, s2, s3, sidx):
    _emb_body(x_hbm, wte_hbm, wpe_hbm, out_hbm, pos_v,
              (t0, t1, t2, t3), idx_v,
              (g0, g1, g2, g3),
              (s0, s1, s2, s3), sidx)


@jax.jit
def _emb(x_flat, wte, wpe):
    mesh = plsc.VectorSubcoreMesh(core_axis_name="c", subcore_axis_name="s")
    return pl.kernel(
        _emb_wrapped,
        out_type=jax.ShapeDtypeStruct((_ROWS, _E), jnp.float32),
        mesh=mesh,
        scratch_types=(
            [pltpu.VMEM((_PC, _E), jnp.float32)]
            + [pltpu.VMEM((_C, _E), jnp.float32) for _ in range(_NB)]
            + [pltpu.VMEM((_B * _TPW,), jnp.int32)]
            + [pltpu.SemaphoreType.DMA for _ in range(2 * _NB + 1)]
        ),
    )(x_flat, wte, wpe)


def kernel(x, wte, wpe, pos):
    del pos
    out = _emb(x.reshape(_ROWS).astype(jnp.int32), wte, wpe)
    return out.reshape(_B, _CW, _E)


# speedup vs baseline: 1.5033x; 1.0235x over previous
import jax
import jax.numpy as jnp
from jax import lax
from jax.experimental import pallas as pl
from jax.experimental.pallas import tpu as pltpu
from jax.experimental.pallas import tpu_sc as plsc

_INFO = plsc.get_sparse_core_info()
_NC, _NS = _INFO.num_cores, _INFO.num_subcores
_NW = _NC * _NS

_B = 4
_CW = 2048
_E = 1024
_ROWS = _B * _CW
_TPW = _CW // _NW
_C = 16
_NB = 4
_NCHUNK = (_TPW // _C) * _B
_PC = 32
_LANE = 16
_COLS = _E // _LANE


def _emb_body(x_hbm, wte_hbm, wpe_hbm, out_hbm,
              pos_v, toks, idx_v, gsems, ssems, psem, sidx):
    wid = lax.axis_index("s") * _NC + lax.axis_index("c")
    t0 = wid * _TPW

    def out_off(i):
        return (i & 3) * _CW + t0 + (i >> 2) * _C

    def idx_off(i):
        return (i & 3) * _TPW + (i >> 2) * _C

    descs = [
        pltpu.async_copy(
            x_hbm.at[pl.ds(b * _CW + t0, _TPW)],
            idx_v.at[pl.ds(b * _TPW, _TPW)],
            sidx,
        )
        for b in range(_B)
    ]
    for d in descs:
        d.wait()

    def start_gather(i, s):
        pltpu.async_copy(
            wte_hbm.at[idx_v.at[pl.ds(idx_off(i), _C)]], toks[s], gsems[s])

    _AH = 2

    start_gather(0, 0)
    start_gather(1, 1)
    pltpu.sync_copy(wpe_hbm.at[pl.ds(t0, _PC)], pos_v)

    def group(g, carry):
        for s in range(_NB):
            i = _NB * g + s
            nxt = (s + _AH) % _NB
            @pl.when(i >= _AH)
            def _():
                for half in range(2):
                    pltpu.make_async_copy(
                        toks[nxt].at[pl.ds(half * (_C // 2), _C // 2)],
                        out_hbm.at[pl.ds(
                            out_off(i - _AH) + half * (_C // 2), _C // 2)],
                        ssems[nxt],
                    ).wait()

            @pl.when(i <= _NCHUNK - 1 - _AH)
            def _():
                start_gather(i + _AH, nxt)

            pltpu.make_async_copy(
                wte_hbm.at[idx_v.at[pl.ds(idx_off(i), _C)]], toks[s],
                gsems[s]).wait()

            @pl.when(i == _NCHUNK // 2 - 2)
            def _():
                pltpu.async_copy(
                    wpe_hbm.at[pl.ds(t0 + _PC, _C)],
                    pos_v.at[pl.ds(0, _C)], psem)

            @pl.when(i == _NCHUNK // 2)
            def _():
                pltpu.make_async_copy(
                    wpe_hbm.at[pl.ds(t0 + _PC, _C)],
                    pos_v.at[pl.ds(0, _C)], psem).wait()
                pltpu.async_copy(
                    wpe_hbm.at[pl.ds(t0 + _PC + _C, _C)],
                    pos_v.at[pl.ds(_C, _C)], psem)

            @pl.when(i == _NCHUNK * 3 // 4)
            def _():
                pltpu.make_async_copy(
                    wpe_hbm.at[pl.ds(t0 + _PC + _C, _C)],
                    pos_v.at[pl.ds(_C, _C)], psem).wait()

            tok = toks[s]
            pbase = (lax.shift_right_logical(i, 2) & (_PC // _C - 1)) * _C

            for half in range(2):
                @plsc.parallel_loop(0, (_C // 2) * _COLS, unroll=16)
                def add_k(k, tok=tok, pbase=pbase, half=half):
                    r = lax.shift_right_logical(k, 6) + half * (_C // 2)
                    j = (k & (_COLS - 1)) * _LANE
                    plsc.addupdate(tok.at[r, pl.ds(j, _LANE)],
                                   pos_v[pbase + r, pl.ds(j, _LANE)])

                pltpu.async_copy(
                    tok.at[pl.ds(half * (_C // 2), _C // 2)],
                    out_hbm.at[pl.ds(out_off(i) + half * (_C // 2), _C // 2)],
                    ssems[s])
        return carry

    lax.fori_loop(0, _NCHUNK // _NB, group, 0)
    for i in range(_NCHUNK - _AH, _NCHUNK):
        for half in range(2):
            pltpu.make_async_copy(
                toks[i % _NB].at[pl.ds(half * (_C // 2), _C // 2)],
                out_hbm.at[pl.ds(out_off(i) + half * (_C // 2), _C // 2)],
                ssems[i % _NB],
            ).wait()


def _emb_wrapped(x_hbm, wte_hbm, wpe_hbm, out_hbm, pos_v,
                 t0, t1, t2, t3, idx_v,
                 g0, g1, g2, g3,
                 s0, s1, s2, s3, psem, sidx):
    _emb_body(x_hbm, wte_hbm, wpe_hbm, out_hbm, pos_v,
              (t0, t1, t2, t3), idx_v,
              (g0, g1, g2, g3),
              (s0, s1, s2, s3), psem, sidx)


@jax.jit
def _emb(x_flat, wte, wpe):
    mesh = plsc.VectorSubcoreMesh(core_axis_name="c", subcore_axis_name="s")
    return pl.kernel(
        _emb_wrapped,
        out_type=jax.ShapeDtypeStruct((_ROWS, _E), jnp.float32),
        mesh=mesh,
        scratch_types=(
            [pltpu.VMEM((_PC, _E), jnp.float32)]
            + [pltpu.VMEM((_C, _E), jnp.float32) for _ in range(_NB)]
            + [pltpu.VMEM((_B * _TPW,), jnp.int32)]
            + [pltpu.SemaphoreType.DMA for _ in range(2 * _NB + 2)]
        ),
    )(x_flat, wte, wpe)


def kernel(x, wte, wpe, pos):
    del pos
    out = _emb(x.reshape(_ROWS).astype(jnp.int32), wte, wpe)
    return out.reshape(_B, _CW, _E)
